# baseline (speedup 1.0000x reference)
import jax
import jax.numpy as jnp
from jax import lax
from jax.experimental import pallas as pl
from jax.experimental.pallas import tpu as pltpu
from jax.experimental.pallas import tpu_sc as plsc

N = 10000
E = 320000
D = 128
T = 4

NC = 2
NS = 16
NW = NC * NS

EDGE_K = 1024
E_PAD = 327680
N_ACC = 10240
ROWS_PER_TILE = N_ACC // NS
WINDOWS = E_PAD // EDGE_K
WPW = WINDOWS // NW
CHUNK = 128
CPW = EDGE_K // CHUNK


def _deg_body(dst2d, zeros_t, ones_l, deg_out,
              didx0, didx1, ones_v, deg_sp, si0, si1, ss0, ss1):
    c = lax.axis_index("c")
    s = lax.axis_index("s")
    w = s * NC + c
    sl = pl.ds(s * ROWS_PER_TILE, ROWS_PER_TILE)
    didx = (didx0, didx1)
    smi = (si0, si1)
    sms = (ss0, ss1)

    pltpu.sync_copy(zeros_t, deg_sp.at[sl])
    pltpu.sync_copy(ones_l, ones_v)
    plsc.subcore_barrier()

    idx_d = [None, None]
    sca_d = [[], []]

    def issue_idx(Wn):
        slot = Wn % 2
        win = w + NW * Wn
        idx_d[slot] = pltpu.async_copy(
            dst2d.at[pl.ds(win * 8, 8)], didx[slot], smi[slot])

    issue_idx(0)
    for Wn in range(WPW):
        slot = Wn % 2
        idx_d[slot].wait()
        if Wn + 1 < WPW:
            for d in sca_d[(Wn + 1) % 2]:
                d.wait()
            sca_d[(Wn + 1) % 2] = []
            issue_idx(Wn + 1)
        for j in range(CPW):
            sca_d[slot].append(pltpu.async_copy(
                ones_v, deg_sp.at[didx[slot].at[j]], sms[slot], add=True))
    for slot in range(2):
        for d in sca_d[slot]:
            d.wait()
    plsc.subcore_barrier()

    @pl.when(s == 0)
    def _():
        pltpu.sync_copy(deg_sp, deg_out.at[c])


def _deg_kernel(dst2d, zeros_t1, ones_l):
    return pl.kernel(
        _deg_body,
        out_type=jax.ShapeDtypeStruct((NC, N_ACC), jnp.float32),
        mesh=plsc.VectorSubcoreMesh(core_axis_name="c", subcore_axis_name="s"),
        scratch_types=[
            pltpu.VMEM((8, 128), jnp.int32),
            pltpu.VMEM((8, 128), jnp.int32),
            pltpu.VMEM((128,), jnp.float32),
            pltpu.VMEM_SHARED((N_ACC,), jnp.float32),
            pltpu.SemaphoreType.DMA,
            pltpu.SemaphoreType.DMA,
            pltpu.SemaphoreType.DMA,
            pltpu.SemaphoreType.DMA,
        ],
    )(dst2d, zeros_t1, ones_l)


def _scatter_body(src_e, dst2d, g_hbm, zeros_t, p_out,
                  sidx0, sidx1, didx0, didx1, rows0, rows1, acc_sp,
                  si0, si1, sg0, sg1, ss0, ss1):
    c = lax.axis_index("c")
    s = lax.axis_index("s")
    w = s * NC + c
    sl = pl.ds(s * ROWS_PER_TILE, ROWS_PER_TILE)
    sidx = (sidx0, sidx1)
    didx = (didx0, didx1)
    rows = (rows0, rows1)
    smi = (si0, si1)
    smg = (sg0, sg1)
    sms = (ss0, ss1)

    pltpu.sync_copy(zeros_t, acc_sp.at[sl])
    plsc.subcore_barrier()

    idx_d = [None, None]

    def issue_idx(Wn):
        slot = Wn % 2
        win = w + NW * Wn
        d1 = pltpu.async_copy(
            src_e.at[pl.ds(win * EDGE_K, EDGE_K)], sidx[slot], smi[slot])
        d2 = pltpu.async_copy(
            dst2d.at[pl.ds(win * 8, 8)], didx[slot], smi[slot])
        idx_d[slot] = (d1, d2)

    def chunk_refs(t):
        slot = (t // CPW) % 2
        sub = t % CPW
        return (sidx[slot].at[pl.ds(sub * CHUNK, CHUNK)], didx[slot].at[sub])

    total = WPW * CPW
    gat = [None, None]
    sca = [None, None]

    issue_idx(0)
    for d in idx_d[0]:
        d.wait()
    sref0, _ = chunk_refs(0)
    gat[0] = pltpu.async_copy(g_hbm.at[sref0], rows[0], smg[0])
    for t in range(total):
        b = t % 2
        Wn, sub = t // CPW, t % CPW
        if sub == 0 and Wn > 0:
            for d in idx_d[Wn % 2]:
                d.wait()
        gat[b].wait()
        if t + 1 < total:
            nb = (t + 1) % 2
            if sca[nb] is not None:
                sca[nb].wait()
            sref, _ = chunk_refs(t + 1)
            gat[nb] = pltpu.async_copy(g_hbm.at[sref], rows[nb], smg[nb])
        _, dref = chunk_refs(t)
        sca[b] = pltpu.async_copy(rows[b], acc_sp.at[dref], sms[b], add=True)
        if sub == 1 and Wn + 1 < WPW:
            issue_idx(Wn + 1)
    sca[0].wait()
    sca[1].wait()
    plsc.subcore_barrier()
    pltpu.sync_copy(acc_sp.at[sl], p_out.at[c, sl])


def _scatter_kernel(src_e, dst2d, g, zeros_t):
    return pl.kernel(
        _scatter_body,
        out_type=jax.ShapeDtypeStruct((NC, N_ACC, D), jnp.float32),
        mesh=plsc.VectorSubcoreMesh(core_axis_name="c", subcore_axis_name="s"),
        scratch_types=[
            pltpu.VMEM((EDGE_K,), jnp.int32),
            pltpu.VMEM((EDGE_K,), jnp.int32),
            pltpu.VMEM((8, 128), jnp.int32),
            pltpu.VMEM((8, 128), jnp.int32),
            pltpu.VMEM((CHUNK, D), jnp.float32),
            pltpu.VMEM((CHUNK, D), jnp.float32),
            pltpu.VMEM_SHARED((N_ACC, D), jnp.float32),
            pltpu.SemaphoreType.DMA,
            pltpu.SemaphoreType.DMA,
            pltpu.SemaphoreType.DMA,
            pltpu.SemaphoreType.DMA,
            pltpu.SemaphoreType.DMA,
            pltpu.SemaphoreType.DMA,
        ],
    )(src_e, dst2d, g, zeros_t)


ROW_BLK = 1024
DEG_BLK = ROW_BLK // 128


def _scale_rows(v, d0, d1):
    deg = 1.0 + d0 + d1
    dinv = lax.rsqrt(jnp.maximum(deg, 1.0))
    v3 = v.reshape(DEG_BLK, 128, D)
    return (v3 * dinv[:, :, None]).reshape(ROW_BLK, D)


def _mm_body(x_ref, w_ref, h_ref):
    h_ref[...] = jnp.dot(x_ref[...], w_ref[...],
                         preferred_element_type=jnp.float32)


def _mm_kernel(x, W):
    return pl.pallas_call(
        _mm_body,
        grid=(pl.cdiv(N, ROW_BLK),),
        in_specs=[
            pl.BlockSpec((ROW_BLK, D), lambda i: (i, 0)),
            pl.BlockSpec((D, D), lambda i: (0, 0)),
        ],
        out_specs=pl.BlockSpec((ROW_BLK, D), lambda i: (i, 0)),
        out_shape=jax.ShapeDtypeStruct((N, D), jnp.float32),
    )(x, W)


def _scale_body(h_ref, d0_ref, d1_ref, g_ref):
    g_ref[...] = _scale_rows(h_ref[...], d0_ref[...], d1_ref[...])


def _scale_kernel(h, d0, d1):
    return pl.pallas_call(
        _scale_body,
        grid=(pl.cdiv(N, ROW_BLK),),
        in_specs=[
            pl.BlockSpec((ROW_BLK, D), lambda i: (i, 0)),
            pl.BlockSpec((DEG_BLK, 128), lambda i: (i, 0)),
            pl.BlockSpec((DEG_BLK, 128), lambda i: (i, 0)),
        ],
        out_specs=pl.BlockSpec((ROW_BLK, D), lambda i: (i, 0)),
        out_shape=jax.ShapeDtypeStruct((N, D), jnp.float32),
    )(h, d0, d1)


def _fin_body(p_ref, g_ref, d0_ref, d1_ref, b_ref, rep_ref, z_ref):
    s = p_ref[0] + p_ref[1] + g_ref[...]
    hf = _scale_rows(s, d0_ref[...], d1_ref[...]) + b_ref[...]
    nrm = jnp.sqrt(jnp.sum(hf * hf, axis=-1, keepdims=True))
    nrm = jnp.maximum(nrm, 1e-15)
    z_ref[...] = jnp.tanh(nrm) * hf / nrm
    rep_ref[...] = jnp.broadcast_to(hf[None], (T,) + hf.shape)


def _fin_kernel(p, g, d0, d1, b2d):
    blk = lambda i: (i, 0)
    return pl.pallas_call(
        _fin_body,
        grid=(pl.cdiv(N, ROW_BLK),),
        in_specs=[
            pl.BlockSpec((NC, ROW_BLK, D), lambda i: (0, i, 0)),
            pl.BlockSpec((ROW_BLK, D), blk),
            pl.BlockSpec((DEG_BLK, 128), blk),
            pl.BlockSpec((DEG_BLK, 128), blk),
            pl.BlockSpec((1, D), lambda i: (0, 0)),
        ],
        out_specs=[
            pl.BlockSpec((T, ROW_BLK, D), lambda i: (0, i, 0)),
            pl.BlockSpec((ROW_BLK, D), blk),
        ],
        out_shape=[
            jax.ShapeDtypeStruct((T, N, D), jnp.float32),
            jax.ShapeDtypeStruct((N, D), jnp.float32),
        ],
    )(p, g, d0, d1, b2d)


def kernel(x, edge_index, W, b):
    npad = E_PAD - E
    pad_src = (jnp.arange(npad, dtype=jnp.int32) * 37) % N
    pad_dst = N + (jnp.arange(npad, dtype=jnp.int32) % (N_ACC - N))
    src = jnp.concatenate([edge_index[0], pad_src])
    dst2d = jnp.concatenate([edge_index[1], pad_dst]).reshape(E_PAD // 128, 128)

    zeros_t1 = jnp.zeros((ROWS_PER_TILE,), jnp.float32)
    ones_l = jnp.ones((128,), jnp.float32)
    zeros_t = jnp.zeros((ROWS_PER_TILE, D), jnp.float32)

    h = _mm_kernel(x, W)
    deg = _deg_kernel(dst2d, zeros_t1, ones_l)
    d0 = deg[0].reshape(N_ACC // 128, 128)
    d1 = deg[1].reshape(N_ACC // 128, 128)

    g = _scale_kernel(h, d0, d1)
    p = _scatter_kernel(src, dst2d, g, zeros_t)

    b2d = b.reshape(1, D)
    x_rep, z = _fin_kernel(p, g, d0, d1, b2d)
    return (x_rep, z)


# speedup vs baseline: 1.1396x; 1.1396x over previous
import jax
import jax.numpy as jnp
from jax import lax
from jax.experimental import pallas as pl
from jax.experimental.pallas import tpu as pltpu
from jax.experimental.pallas import tpu_sc as plsc

N = 10000
E = 320000
D = 128
T = 4

NC = 2
NS = 16
NW = NC * NS

EDGE_K = 1024
E_PAD = 327680
N_ACC = 10240
ROWS_PER_TILE = N_ACC // NS
WINDOWS = E_PAD // EDGE_K
WPW = WINDOWS // NW
CHUNK = 128
CPW = EDGE_K // CHUNK


def _deg_body(dst2d, zeros_t, ones_l, deg_out,
              didx0, didx1, ones_v, deg_sp, si0, si1, ss0, ss1):
    c = lax.axis_index("c")
    s = lax.axis_index("s")
    w = s * NC + c
    sl = pl.ds(s * ROWS_PER_TILE, ROWS_PER_TILE)
    didx = (didx0, didx1)
    smi = (si0, si1)
    sms = (ss0, ss1)

    pltpu.sync_copy(zeros_t, deg_sp.at[sl])
    pltpu.sync_copy(ones_l, ones_v)
    plsc.subcore_barrier()

    idx_d = [None, None]
    sca_d = [[], []]

    def issue_idx(Wn):
        slot = Wn % 2
        win = w + NW * Wn
        idx_d[slot] = pltpu.async_copy(
            dst2d.at[pl.ds(win * 8, 8)], didx[slot], smi[slot])

    issue_idx(0)
    for Wn in range(WPW):
        slot = Wn % 2
        idx_d[slot].wait()
        if Wn + 1 < WPW:
            for d in sca_d[(Wn + 1) % 2]:
                d.wait()
            sca_d[(Wn + 1) % 2] = []
            issue_idx(Wn + 1)
        for j in range(CPW):
            sca_d[slot].append(pltpu.async_copy(
                ones_v, deg_sp.at[didx[slot].at[j]], sms[slot], add=True))
    for slot in range(2):
        for d in sca_d[slot]:
            d.wait()
    plsc.subcore_barrier()

    @pl.when(s == 0)
    def _():
        pltpu.sync_copy(deg_sp, deg_out.at[c])


def _deg_kernel(dst2d, zeros_t1, ones_l):
    return pl.kernel(
        _deg_body,
        out_type=jax.ShapeDtypeStruct((NC, N_ACC), jnp.float32),
        mesh=plsc.VectorSubcoreMesh(core_axis_name="c", subcore_axis_name="s"),
        scratch_types=[
            pltpu.VMEM((8, 128), jnp.int32),
            pltpu.VMEM((8, 128), jnp.int32),
            pltpu.VMEM((128,), jnp.float32),
            pltpu.VMEM_SHARED((N_ACC,), jnp.float32),
            pltpu.SemaphoreType.DMA,
            pltpu.SemaphoreType.DMA,
            pltpu.SemaphoreType.DMA,
            pltpu.SemaphoreType.DMA,
        ],
    )(dst2d, zeros_t1, ones_l)


def _scatter_body(src_e, dst2d, g_hbm, zeros_t, p_out,
                  sidx0, sidx1, didx0, didx1, rows0, rows1, acc_sp,
                  si0, si1, sg0, sg1, ss0, ss1):
    c = lax.axis_index("c")
    s = lax.axis_index("s")
    w = s * NC + c
    sl = pl.ds(s * ROWS_PER_TILE, ROWS_PER_TILE)
    sidx = (sidx0, sidx1)
    didx = (didx0, didx1)
    rows = (rows0, rows1)
    smi = (si0, si1)
    smg = (sg0, sg1)
    sms = (ss0, ss1)

    pltpu.sync_copy(zeros_t, acc_sp.at[sl])
    plsc.subcore_barrier()

    idx_d = [None, None]

    def issue_idx(Wn):
        slot = Wn % 2
        win = w + NW * Wn
        d1 = pltpu.async_copy(
            src_e.at[pl.ds(win * EDGE_K, EDGE_K)], sidx[slot], smi[slot])
        d2 = pltpu.async_copy(
            dst2d.at[pl.ds(win * 8, 8)], didx[slot], smi[slot])
        idx_d[slot] = (d1, d2)

    def chunk_refs(t):
        slot = (t // CPW) % 2
        sub = t % CPW
        return (sidx[slot].at[pl.ds(sub * CHUNK, CHUNK)], didx[slot].at[sub])

    total = WPW * CPW
    gat = [None, None]
    sca = [None, None]

    issue_idx(0)
    for d in idx_d[0]:
        d.wait()
    sref0, _ = chunk_refs(0)
    gat[0] = pltpu.async_copy(g_hbm.at[sref0], rows[0], smg[0])
    for t in range(total):
        b = t % 2
        Wn, sub = t // CPW, t % CPW
        if t + 1 < total:
            nb = (t + 1) % 2
            if sca[nb] is not None:
                sca[nb].wait()
            if (t + 1) // CPW != Wn:
                for d in idx_d[(Wn + 1) % 2]:
                    d.wait()
            sref, _ = chunk_refs(t + 1)
            gat[nb] = pltpu.async_copy(g_hbm.at[sref], rows[nb], smg[nb])
        gat[b].wait()
        _, dref = chunk_refs(t)
        sca[b] = pltpu.async_copy(rows[b], acc_sp.at[dref], sms[b], add=True)
        if sub == 1 and Wn + 1 < WPW:
            issue_idx(Wn + 1)
    sca[0].wait()
    sca[1].wait()
    plsc.subcore_barrier()
    pltpu.sync_copy(acc_sp.at[sl], p_out.at[c, sl])


def _scatter_kernel(src_e, dst2d, g, zeros_t):
    return pl.kernel(
        _scatter_body,
        out_type=jax.ShapeDtypeStruct((NC, N_ACC, D), jnp.float32),
        mesh=plsc.VectorSubcoreMesh(core_axis_name="c", subcore_axis_name="s"),
        scratch_types=[
            pltpu.VMEM((EDGE_K,), jnp.int32),
            pltpu.VMEM((EDGE_K,), jnp.int32),
            pltpu.VMEM((8, 128), jnp.int32),
            pltpu.VMEM((8, 128), jnp.int32),
            pltpu.VMEM((CHUNK, D), jnp.float32),
            pltpu.VMEM((CHUNK, D), jnp.float32),
            pltpu.VMEM_SHARED((N_ACC, D), jnp.float32),
            pltpu.SemaphoreType.DMA,
            pltpu.SemaphoreType.DMA,
            pltpu.SemaphoreType.DMA,
            pltpu.SemaphoreType.DMA,
            pltpu.SemaphoreType.DMA,
            pltpu.SemaphoreType.DMA,
        ],
    )(src_e, dst2d, g, zeros_t)


ROW_BLK = 1024
DEG_BLK = ROW_BLK // 128


def _scale_rows(v, d0, d1):
    deg = 1.0 + d0 + d1
    dinv = lax.rsqrt(jnp.maximum(deg, 1.0))
    v3 = v.reshape(DEG_BLK, 128, D)
    return (v3 * dinv[:, :, None]).reshape(ROW_BLK, D)


def _gw_body(x_ref, w_ref, d0_ref, d1_ref, g_ref):
    h = jnp.dot(x_ref[...], w_ref[...], preferred_element_type=jnp.float32)
    g_ref[...] = _scale_rows(h, d0_ref[...], d1_ref[...])


def _gw_kernel(x, W, d0, d1):
    return pl.pallas_call(
        _gw_body,
        grid=(pl.cdiv(N, ROW_BLK),),
        in_specs=[
            pl.BlockSpec((ROW_BLK, D), lambda i: (i, 0)),
            pl.BlockSpec((D, D), lambda i: (0, 0)),
            pl.BlockSpec((DEG_BLK, 128), lambda i: (i, 0)),
            pl.BlockSpec((DEG_BLK, 128), lambda i: (i, 0)),
        ],
        out_specs=pl.BlockSpec((ROW_BLK, D), lambda i: (i, 0)),
        out_shape=jax.ShapeDtypeStruct((N, D), jnp.float32),
    )(x, W, d0, d1)


def _fin_body(p_ref, g_ref, d0_ref, d1_ref, b_ref, rep_ref, z_ref):
    s = p_ref[0] + p_ref[1] + g_ref[...]
    hf = _scale_rows(s, d0_ref[...], d1_ref[...]) + b_ref[...]
    nrm = jnp.sqrt(jnp.sum(hf * hf, axis=-1, keepdims=True))
    nrm = jnp.maximum(nrm, 1e-15)
    z_ref[...] = jnp.tanh(nrm) * hf / nrm
    rep_ref[...] = jnp.broadcast_to(hf[None], (T,) + hf.shape)


def _fin_kernel(p, g, d0, d1, b2d):
    blk = lambda i: (i, 0)
    return pl.pallas_call(
        _fin_body,
        grid=(pl.cdiv(N, ROW_BLK),),
        in_specs=[
            pl.BlockSpec((NC, ROW_BLK, D), lambda i: (0, i, 0)),
            pl.BlockSpec((ROW_BLK, D), blk),
            pl.BlockSpec((DEG_BLK, 128), blk),
            pl.BlockSpec((DEG_BLK, 128), blk),
            pl.BlockSpec((1, D), lambda i: (0, 0)),
        ],
        out_specs=[
            pl.BlockSpec((T, ROW_BLK, D), lambda i: (0, i, 0)),
            pl.BlockSpec((ROW_BLK, D), blk),
        ],
        out_shape=[
            jax.ShapeDtypeStruct((T, N, D), jnp.float32),
            jax.ShapeDtypeStruct((N, D), jnp.float32),
        ],
    )(p, g, d0, d1, b2d)


def kernel(x, edge_index, W, b):
    npad = E_PAD - E
    pad_src = (jnp.arange(npad, dtype=jnp.int32) * 37) % N
    pad_dst = N + (jnp.arange(npad, dtype=jnp.int32) % (N_ACC - N))
    src = jnp.concatenate([edge_index[0], pad_src])
    dst2d = jnp.concatenate([edge_index[1], pad_dst]).reshape(E_PAD // 128, 128)

    zeros_t1 = jnp.zeros((ROWS_PER_TILE,), jnp.float32)
    ones_l = jnp.ones((128,), jnp.float32)
    zeros_t = jnp.zeros((ROWS_PER_TILE, D), jnp.float32)

    deg = _deg_kernel(dst2d, zeros_t1, ones_l)
    d0 = deg[0].reshape(N_ACC // 128, 128)
    d1 = deg[1].reshape(N_ACC // 128, 128)

    g = _gw_kernel(x, W, d0, d1)
    p = _scatter_kernel(src, dst2d, g, zeros_t)

    b2d = b.reshape(1, D)
    x_rep, z = _fin_kernel(p, g, d0, d1, b2d)
    return (x_rep, z)


# speedup vs baseline: 1.1449x; 1.0046x over previous
import jax
import jax.numpy as jnp
from jax import lax
from jax.experimental import pallas as pl
from jax.experimental.pallas import tpu as pltpu
from jax.experimental.pallas import tpu_sc as plsc

N = 10000
E = 320000
D = 128
T = 4

NC = 2
NS = 16
NW = NC * NS

EDGE_K = 1024
E_PAD = 327680
N_ACC = 10240
ROWS_PER_TILE = N_ACC // NS
WINDOWS = E_PAD // EDGE_K
WPW = WINDOWS // NW
CHUNK = 128
CPW = EDGE_K // CHUNK


def _deg_body(dst2d, zeros_t, ones_l, deg_out,
              didx0, didx1, ones_v, deg_sp, si0, si1, ss0, ss1):
    c = lax.axis_index("c")
    s = lax.axis_index("s")
    w = s * NC + c
    sl = pl.ds(s * ROWS_PER_TILE, ROWS_PER_TILE)
    didx = (didx0, didx1)
    smi = (si0, si1)
    sms = (ss0, ss1)

    pltpu.sync_copy(zeros_t, deg_sp.at[sl])
    pltpu.sync_copy(ones_l, ones_v)
    plsc.subcore_barrier()

    idx_d = [None, None]
    sca_d = [[], []]

    def issue_idx(Wn):
        slot = Wn % 2
        win = w + NW * Wn
        idx_d[slot] = pltpu.async_copy(
            dst2d.at[pl.ds(win * 8, 8)], didx[slot], smi[slot])

    issue_idx(0)
    for Wn in range(WPW):
        slot = Wn % 2
        idx_d[slot].wait()
        if Wn + 1 < WPW:
            for d in sca_d[(Wn + 1) % 2]:
                d.wait()
            sca_d[(Wn + 1) % 2] = []
            issue_idx(Wn + 1)
        for j in range(CPW):
            sca_d[slot].append(pltpu.async_copy(
                ones_v, deg_sp.at[didx[slot].at[j]], sms[slot], add=True))
    for slot in range(2):
        for d in sca_d[slot]:
            d.wait()
    plsc.subcore_barrier()

    @pl.when(s == 0)
    def _():
        pltpu.sync_copy(deg_sp, deg_out.at[c])


def _deg_kernel(dst2d, zeros_t1, ones_l):
    return pl.kernel(
        _deg_body,
        out_type=jax.ShapeDtypeStruct((NC, N_ACC), jnp.float32),
        mesh=plsc.VectorSubcoreMesh(core_axis_name="c", subcore_axis_name="s"),
        scratch_types=[
            pltpu.VMEM((8, 128), jnp.int32),
            pltpu.VMEM((8, 128), jnp.int32),
            pltpu.VMEM((128,), jnp.float32),
            pltpu.VMEM_SHARED((N_ACC,), jnp.float32),
            pltpu.SemaphoreType.DMA,
            pltpu.SemaphoreType.DMA,
            pltpu.SemaphoreType.DMA,
            pltpu.SemaphoreType.DMA,
        ],
    )(dst2d, zeros_t1, ones_l)


def _scatter_body(src_e, dst2d, g_hbm, zeros_t, p_out,
                  sidx0, sidx1, didx0, didx1, rows0, rows1, acc_sp,
                  si0, si1, sg0, sg1, ss0, ss1):
    c = lax.axis_index("c")
    s = lax.axis_index("s")
    w = s * NC + c
    sl = pl.ds(s * ROWS_PER_TILE, ROWS_PER_TILE)
    sidx = (sidx0, sidx1)
    didx = (didx0, didx1)
    rows = (rows0, rows1)
    smi = (si0, si1)
    smg = (sg0, sg1)
    sms = (ss0, ss1)

    pltpu.sync_copy(zeros_t, acc_sp.at[sl])
    plsc.subcore_barrier()

    idx_d = [None, None]

    def issue_idx(Wn):
        slot = Wn % 2
        win = w + NW * Wn
        d1 = pltpu.async_copy(
            src_e.at[pl.ds(win * EDGE_K, EDGE_K)], sidx[slot], smi[slot])
        d2 = pltpu.async_copy(
            dst2d.at[pl.ds(win * 8, 8)], didx[slot], smi[slot])
        idx_d[slot] = (d1, d2)

    def chunk_refs(t):
        slot = (t // CPW) % 2
        sub = t % CPW
        return (sidx[slot].at[pl.ds(sub * CHUNK, CHUNK)], didx[slot].at[sub])

    def issue_gather(t, nb):
        slot = (t // CPW) % 2
        sub = t % CPW
        h = CHUNK // 2
        d1 = pltpu.async_copy(
            g_hbm.at[sidx[slot].at[pl.ds(sub * CHUNK, h)]],
            rows[nb].at[pl.ds(0, h)], smg[nb])
        d2 = pltpu.async_copy(
            g_hbm.at[sidx[slot].at[pl.ds(sub * CHUNK + h, h)]],
            rows[nb].at[pl.ds(h, h)], smg[nb])
        return (d1, d2)

    total = WPW * CPW
    gat = [None, None]
    sca = [None, None]

    issue_idx(0)
    for d in idx_d[0]:
        d.wait()
    gat[0] = issue_gather(0, 0)
    for t in range(total):
        b = t % 2
        Wn, sub = t // CPW, t % CPW
        if t + 1 < total:
            nb = (t + 1) % 2
            if sca[nb] is not None:
                sca[nb].wait()
            if (t + 1) // CPW != Wn:
                for d in idx_d[(Wn + 1) % 2]:
                    d.wait()
            gat[nb] = issue_gather(t + 1, nb)
        for d in gat[b]:
            d.wait()
        _, dref = chunk_refs(t)
        sca[b] = pltpu.async_copy(rows[b], acc_sp.at[dref], sms[b], add=True)
        if sub == 1 and Wn + 1 < WPW:
            issue_idx(Wn + 1)
    sca[0].wait()
    sca[1].wait()
    plsc.subcore_barrier()
    pltpu.sync_copy(acc_sp.at[sl], p_out.at[c, sl])


def _scatter_kernel(src_e, dst2d, g, zeros_t):
    return pl.kernel(
        _scatter_body,
        out_type=jax.ShapeDtypeStruct((NC, N_ACC, D), jnp.float32),
        mesh=plsc.VectorSubcoreMesh(core_axis_name="c", subcore_axis_name="s"),
        scratch_types=[
            pltpu.VMEM((EDGE_K,), jnp.int32),
            pltpu.VMEM((EDGE_K,), jnp.int32),
            pltpu.VMEM((8, 128), jnp.int32),
            pltpu.VMEM((8, 128), jnp.int32),
            pltpu.VMEM((CHUNK, D), jnp.float32),
            pltpu.VMEM((CHUNK, D), jnp.float32),
            pltpu.VMEM_SHARED((N_ACC, D), jnp.float32),
            pltpu.SemaphoreType.DMA,
            pltpu.SemaphoreType.DMA,
            pltpu.SemaphoreType.DMA,
            pltpu.SemaphoreType.DMA,
            pltpu.SemaphoreType.DMA,
            pltpu.SemaphoreType.DMA,
        ],
    )(src_e, dst2d, g, zeros_t)


ROW_BLK = 1024
DEG_BLK = ROW_BLK // 128


def _scale_rows(v, d0, d1):
    deg = 1.0 + d0 + d1
    dinv = lax.rsqrt(jnp.maximum(deg, 1.0))
    v3 = v.reshape(DEG_BLK, 128, D)
    return (v3 * dinv[:, :, None]).reshape(ROW_BLK, D)


def _gw_body(x_ref, w_ref, d0_ref, d1_ref, g_ref):
    h = jnp.dot(x_ref[...], w_ref[...], preferred_element_type=jnp.float32)
    g_ref[...] = _scale_rows(h, d0_ref[...], d1_ref[...])


def _gw_kernel(x, W, d0, d1):
    return pl.pallas_call(
        _gw_body,
        grid=(pl.cdiv(N, ROW_BLK),),
        in_specs=[
            pl.BlockSpec((ROW_BLK, D), lambda i: (i, 0)),
            pl.BlockSpec((D, D), lambda i: (0, 0)),
            pl.BlockSpec((DEG_BLK, 128), lambda i: (i, 0)),
            pl.BlockSpec((DEG_BLK, 128), lambda i: (i, 0)),
        ],
        out_specs=pl.BlockSpec((ROW_BLK, D), lambda i: (i, 0)),
        out_shape=jax.ShapeDtypeStruct((N, D), jnp.float32),
    )(x, W, d0, d1)


def _fin_body(p_ref, g_ref, d0_ref, d1_ref, b_ref, rep_ref, z_ref):
    s = p_ref[0] + p_ref[1] + g_ref[...]
    hf = _scale_rows(s, d0_ref[...], d1_ref[...]) + b_ref[...]
    nrm = jnp.sqrt(jnp.sum(hf * hf, axis=-1, keepdims=True))
    nrm = jnp.maximum(nrm, 1e-15)
    z_ref[...] = jnp.tanh(nrm) * hf / nrm
    rep_ref[...] = jnp.broadcast_to(hf[None], (T,) + hf.shape)


def _fin_kernel(p, g, d0, d1, b2d):
    blk = lambda i: (i, 0)
    return pl.pallas_call(
        _fin_body,
        grid=(pl.cdiv(N, ROW_BLK),),
        in_specs=[
            pl.BlockSpec((NC, ROW_BLK, D), lambda i: (0, i, 0)),
            pl.BlockSpec((ROW_BLK, D), blk),
            pl.BlockSpec((DEG_BLK, 128), blk),
            pl.BlockSpec((DEG_BLK, 128), blk),
            pl.BlockSpec((1, D), lambda i: (0, 0)),
        ],
        out_specs=[
            pl.BlockSpec((T, ROW_BLK, D), lambda i: (0, i, 0)),
            pl.BlockSpec((ROW_BLK, D), blk),
        ],
        out_shape=[
            jax.ShapeDtypeStruct((T, N, D), jnp.float32),
            jax.ShapeDtypeStruct((N, D), jnp.float32),
        ],
    )(p, g, d0, d1, b2d)


def kernel(x, edge_index, W, b):
    npad = E_PAD - E
    pad_src = (jnp.arange(npad, dtype=jnp.int32) * 37) % N
    pad_dst = N + (jnp.arange(npad, dtype=jnp.int32) % (N_ACC - N))
    src = jnp.concatenate([edge_index[0], pad_src])
    dst2d = jnp.concatenate([edge_index[1], pad_dst]).reshape(E_PAD // 128, 128)

    zeros_t1 = jnp.zeros((ROWS_PER_TILE,), jnp.float32)
    ones_l = jnp.ones((128,), jnp.float32)
    zeros_t = jnp.zeros((ROWS_PER_TILE, D), jnp.float32)

    deg = _deg_kernel(dst2d, zeros_t1, ones_l)
    d0 = deg[0].reshape(N_ACC // 128, 128)
    d1 = deg[1].reshape(N_ACC // 128, 128)

    g = _gw_kernel(x, W, d0, d1)
    p = _scatter_kernel(src, dst2d, g, zeros_t)

    b2d = b.reshape(1, D)
    x_rep, z = _fin_kernel(p, g, d0, d1, b2d)
    return (x_rep, z)
